# baseline (speedup 1.0000x reference)
import functools

import jax
import jax.numpy as jnp
from jax import lax
from jax.experimental import pallas as pl
from jax.experimental.pallas import tpu as pltpu
from jax.experimental.pallas import tpu_sc as plsc

_BS = 16
_NPG = 4096
_C = 256
_NKP = 64
_KNN = 10
_NSEL = _KNN * _NKP
_NROWS = _BS * _NSEL
_PPAD = 16


def _topk_body(feat_ref, posb_ref, pos_t_ref, wr_ref, idx_ref, rel_ref):
    b = pl.program_id(0)
    feat = feat_ref[...].astype(jnp.bfloat16)
    posb = posb_ref[...]
    s = jnp.dot(feat, wr_ref[...], preferred_element_type=jnp.float32)
    m = jnp.max(s, axis=0, keepdims=True)
    p = jnp.exp(s - m)
    pw = p / jnp.sum(p, axis=0, keepdims=True)
    kp = jax.lax.dot_general(pw.astype(jnp.bfloat16), posb,
                             (((0,), (0,)), ((), ())),
                             preferred_element_type=jnp.float32)

    d2 = jnp.zeros((_NKP, _NPG), jnp.float32)
    for d in range(3):
        diff = kp[:, d:d + 1] - pos_t_ref[0, d:d + 1, :]
        d2 = d2 + diff * diff

    iotaf = jax.lax.broadcasted_iota(jnp.int32, (_NKP, _NPG), 1).astype(
        jnp.float32)
    big = jnp.float32(jnp.inf)
    npgf = jnp.float32(_NPG)
    idxs = []
    hits = []
    dmin = jnp.min(d2, axis=1, keepdims=True)
    for r in range(_KNN):
        idx = jnp.min(jnp.where(d2 <= dmin, iotaf, npgf), axis=1, keepdims=True)
        idxs.append(idx)
        hit = iotaf == idx
        hits.append(hit.astype(jnp.bfloat16))
        d2 = jnp.where(hit, big, d2)
        if r + 1 < _KNN:
            dmin = jnp.min(d2, axis=1, keepdims=True)

    idxcol = jnp.concatenate(idxs, axis=0)
    gidx = idxcol.astype(jnp.int32).reshape(1, _NSEL) + b * _NPG
    idx_ref[0] = gidx

    onehot = jnp.concatenate(hits, axis=0)
    pj = jnp.dot(onehot, posb, preferred_element_type=jnp.float32)
    kp_rep = jnp.concatenate([kp] * _KNN, axis=0)
    rel_ref[0] = pj - kp_rep


def _sc_gather(feat_hbm, idx_hbm, outf_hbm, idx_v, rows_v, sem_f):
    nw = 32
    bpw = _NROWS // nw
    wid = lax.axis_index("s") * 2 + lax.axis_index("c")
    base = wid * bpw
    pltpu.sync_copy(idx_hbm.at[pl.ds(base, bpw)], idx_v)
    pltpu.async_copy(feat_hbm.at[idx_v], rows_v, sem_f).wait()
    pltpu.sync_copy(rows_v, outf_hbm.at[pl.ds(base, bpw)])


def _extract_body(g_ref, rel_ref, wef_ref, wep_ref, out_ref):
    fj = g_ref[0].astype(jnp.bfloat16)
    rel = rel_ref[0].astype(jnp.bfloat16)
    ext = (jnp.dot(fj, wef_ref[...], preferred_element_type=jnp.float32)
           + jnp.dot(rel, wep_ref[...], preferred_element_type=jnp.float32))
    ext = jnp.maximum(ext, 0.0).reshape(_KNN, _NKP, _C)
    out_ref[0] = jnp.sum(ext, axis=0) * (1.0 / _KNN)


def kernel(feature, pos, W_pool, W_regress, W_extract):
    del W_pool
    bf = jnp.bfloat16
    posb = pos.astype(bf)
    wr = W_regress[:_C].astype(bf)
    wef = W_extract[:_C].astype(bf)
    wep = W_extract[_C:].astype(bf)
    pos_t = pos.reshape(_BS, _NPG, 3).transpose(0, 2, 1)

    gidx, rel = pl.pallas_call(
        _topk_body,
        grid=(_BS,),
        in_specs=[
            pl.BlockSpec((_NPG, _C), lambda b: (b, 0)),
            pl.BlockSpec((_NPG, 3), lambda b: (b, 0)),
            pl.BlockSpec((1, 3, _NPG), lambda b: (b, 0, 0)),
            pl.BlockSpec((_C, _NKP), lambda b: (0, 0)),
        ],
        out_specs=[
            pl.BlockSpec((1, 1, _NSEL), lambda b: (b, 0, 0)),
            pl.BlockSpec((1, _NSEL, 3), lambda b: (b, 0, 0)),
        ],
        out_shape=[
            jax.ShapeDtypeStruct((_BS, 1, _NSEL), jnp.int32),
            jax.ShapeDtypeStruct((_BS, _NSEL, 3), jnp.float32),
        ],
    )(feature, posb, pos_t, wr)

    idx_flat = gidx.reshape(_NROWS)

    mesh = plsc.VectorSubcoreMesh(core_axis_name="c", subcore_axis_name="s")
    bpw = _NROWS // 32
    sc = functools.partial(
        pl.kernel, mesh=mesh,
        out_type=jax.ShapeDtypeStruct((_NROWS, _C), jnp.float32),
        scratch_types=[
            pltpu.VMEM((bpw,), jnp.int32),
            pltpu.VMEM((bpw, _C), jnp.float32),
            pltpu.SemaphoreType.DMA,
        ],
    )(_sc_gather)
    g_rows = sc(feature, idx_flat)

    g3 = g_rows.reshape(_BS, _NSEL, _C)

    return pl.pallas_call(
        _extract_body,
        grid=(_BS,),
        in_specs=[
            pl.BlockSpec((1, _NSEL, _C), lambda b: (b, 0, 0)),
            pl.BlockSpec((1, _NSEL, 3), lambda b: (b, 0, 0)),
            pl.BlockSpec((_C, _C), lambda b: (0, 0)),
            pl.BlockSpec((3, _C), lambda b: (0, 0)),
        ],
        out_specs=pl.BlockSpec((1, _NKP, _C), lambda b: (b, 0, 0)),
        out_shape=jax.ShapeDtypeStruct((_BS, _NKP, _C), jnp.float32),
    )(g3, rel, wef, wep)


# speedup vs baseline: 1.3138x; 1.3138x over previous
import jax
import jax.numpy as jnp
from jax.experimental import pallas as pl

_BS = 16
_NPG = 4096
_C = 256
_NKP = 64
_KNN = 10


def _lane_min(x):
    w = x.shape[1]
    while w > 128:
        h = w // 2
        x = jnp.minimum(x[:, :h], x[:, h:])
        w = h
    return jnp.min(x, axis=1, keepdims=True)


def _row_max(x):
    r = x.shape[0]
    while r > 8:
        h = r // 2
        x = jnp.maximum(x[:h], x[h:])
        r = h
    return jnp.max(x, axis=0, keepdims=True)


def _body(feat_ref, posb_ref, pos_t_ref, wr_ref, wef_ref, wep_ref, out_ref):
    feat = feat_ref[...].astype(jnp.bfloat16)
    posb = posb_ref[...]
    s = jnp.dot(feat, wr_ref[...], preferred_element_type=jnp.float32)
    m = _row_max(s)
    p = jnp.exp(s - m)
    pw = p / jnp.sum(p, axis=0, keepdims=True)
    kp = jax.lax.dot_general(pw.astype(jnp.bfloat16), posb,
                             (((0,), (0,)), ((), ())),
                             preferred_element_type=jnp.float32)

    d2 = jnp.zeros((_NKP, _NPG), jnp.float32)
    for d in range(3):
        diff = kp[:, d:d + 1] - pos_t_ref[0, d:d + 1, :]
        d2 = d2 + diff * diff

    iot = jax.lax.broadcasted_iota(jnp.int32, (16, _NPG), 1).astype(
        jnp.float32)
    big = jnp.float32(jnp.inf)
    npgf = jnp.float32(_NPG)
    nch = _NKP // 16
    d2s = [d2[c * 16:(c + 1) * 16] for c in range(nch)]
    dmins = [_lane_min(x) for x in d2s]
    hits = []
    for r in range(_KNN):
        for c in range(nch):
            idx = _lane_min(jnp.where(d2s[c] <= dmins[c], iot, npgf))
            hit = iot == idx
            hits.append(hit.astype(jnp.bfloat16))
            d2s[c] = jnp.where(hit, big, d2s[c])
            if r + 1 < _KNN:
                dmins[c] = _lane_min(d2s[c])

    onehot = jnp.concatenate(hits, axis=0)
    fpcat = jnp.concatenate([feat, posb], axis=1)
    g = jnp.dot(onehot, fpcat, preferred_element_type=jnp.float32)
    fj = g[:, :_C].astype(jnp.bfloat16)
    pj = g[:, _C:]
    kp_rep = jnp.concatenate([kp] * _KNN, axis=0)
    rel = (pj - kp_rep).astype(jnp.bfloat16)
    ext = (jnp.dot(fj, wef_ref[...], preferred_element_type=jnp.float32)
           + jnp.dot(rel, wep_ref[...], preferred_element_type=jnp.float32))
    ext = jnp.maximum(ext, 0.0).reshape(_KNN, _NKP, _C)
    out_ref[0] = jnp.sum(ext, axis=0) * (1.0 / _KNN)


def kernel(feature, pos, W_pool, W_regress, W_extract):
    del W_pool
    bf = jnp.bfloat16
    posb = pos.astype(bf)
    wr = W_regress[:_C].astype(bf)
    wef = W_extract[:_C].astype(bf)
    wep = W_extract[_C:].astype(bf)
    pos_t = pos.reshape(_BS, _NPG, 3).transpose(0, 2, 1)
    return pl.pallas_call(
        _body,
        grid=(_BS,),
        in_specs=[
            pl.BlockSpec((_NPG, _C), lambda b: (b, 0)),
            pl.BlockSpec((_NPG, 3), lambda b: (b, 0)),
            pl.BlockSpec((1, 3, _NPG), lambda b: (b, 0, 0)),
            pl.BlockSpec((_C, _NKP), lambda b: (0, 0)),
            pl.BlockSpec((_C, _C), lambda b: (0, 0)),
            pl.BlockSpec((3, _C), lambda b: (0, 0)),
        ],
        out_specs=pl.BlockSpec((1, _NKP, _C), lambda b: (b, 0, 0)),
        out_shape=jax.ShapeDtypeStruct((_BS, _NKP, _C), jnp.float32),
    )(feature, posb, pos_t, wr, wef, wep)


# speedup vs baseline: 1.3352x; 1.0163x over previous
import jax
import jax.numpy as jnp
from jax.experimental import pallas as pl

_BS = 16
_NPG = 4096
_C = 256
_NKP = 64
_KNN = 10


def _lane_min(x):
    w = x.shape[1]
    while w > 128:
        h = w // 2
        x = jnp.minimum(x[:, :h], x[:, h:])
        w = h
    return jnp.min(x, axis=1, keepdims=True)


def _row_max(x):
    r = x.shape[0]
    while r > 8:
        h = r // 2
        x = jnp.maximum(x[:h], x[h:])
        r = h
    return jnp.max(x, axis=0, keepdims=True)


def _row_sum(x):
    r = x.shape[0]
    while r > 8:
        h = r // 2
        x = x[:h] + x[h:]
        r = h
    return jnp.sum(x, axis=0, keepdims=True)


def _body(feat_ref, posb_ref, pos_t_ref, wr_ref, wef_ref, wep_ref, out_ref):
    feat = feat_ref[...].astype(jnp.bfloat16)
    posb = posb_ref[...]
    s = jnp.dot(feat, wr_ref[...], preferred_element_type=jnp.float32)
    m = _row_max(s)
    p = jnp.exp(s - m)
    pw = p / _row_sum(p)
    kp = jax.lax.dot_general(pw.astype(jnp.bfloat16), posb,
                             (((0,), (0,)), ((), ())),
                             preferred_element_type=jnp.float32)

    d2 = jnp.zeros((_NKP, _NPG), jnp.float32)
    for d in range(3):
        diff = kp[:, d:d + 1] - pos_t_ref[0, d:d + 1, :]
        d2 = d2 + diff * diff

    iot = jax.lax.broadcasted_iota(jnp.int32, (8, _NPG), 1).astype(
        jnp.float32)
    big = jnp.float32(jnp.inf)
    npgf = jnp.float32(_NPG)
    nch = _NKP // 8
    d2s = [d2[c * 8:(c + 1) * 8] for c in range(nch)]
    dmins = [_lane_min(x) for x in d2s]
    hits = []
    for r in range(_KNN):
        for c in range(nch):
            idx = _lane_min(jnp.where(d2s[c] <= dmins[c], iot, npgf))
            hit = iot == idx
            hits.append(hit.astype(jnp.bfloat16))
            d2s[c] = jnp.where(hit, big, d2s[c])
            if r + 1 < _KNN:
                dmins[c] = _lane_min(d2s[c])

    onehot = jnp.concatenate(hits, axis=0)
    fpcat = jnp.concatenate([feat, posb], axis=1)
    g = jnp.dot(onehot, fpcat, preferred_element_type=jnp.float32)
    fj = g[:, :_C].astype(jnp.bfloat16)
    pj = g[:, _C:]
    kp_rep = jnp.concatenate([kp] * _KNN, axis=0)
    rel = (pj - kp_rep).astype(jnp.bfloat16)
    ext = (jnp.dot(fj, wef_ref[...], preferred_element_type=jnp.float32)
           + jnp.dot(rel, wep_ref[...], preferred_element_type=jnp.float32))
    ext = jnp.maximum(ext, 0.0).reshape(_KNN, _NKP, _C)
    out_ref[0] = jnp.sum(ext, axis=0) * (1.0 / _KNN)


def kernel(feature, pos, W_pool, W_regress, W_extract):
    del W_pool
    bf = jnp.bfloat16
    posb = pos.astype(bf)
    wr = W_regress[:_C].astype(bf)
    wef = W_extract[:_C].astype(bf)
    wep = W_extract[_C:].astype(bf)
    pos_t = pos.reshape(_BS, _NPG, 3).transpose(0, 2, 1)
    return pl.pallas_call(
        _body,
        grid=(_BS,),
        in_specs=[
            pl.BlockSpec((_NPG, _C), lambda b: (b, 0)),
            pl.BlockSpec((_NPG, 3), lambda b: (b, 0)),
            pl.BlockSpec((1, 3, _NPG), lambda b: (b, 0, 0)),
            pl.BlockSpec((_C, _NKP), lambda b: (0, 0)),
            pl.BlockSpec((_C, _C), lambda b: (0, 0)),
            pl.BlockSpec((3, _C), lambda b: (0, 0)),
        ],
        out_specs=pl.BlockSpec((1, _NKP, _C), lambda b: (b, 0, 0)),
        out_shape=jax.ShapeDtypeStruct((_BS, _NKP, _C), jnp.float32),
    )(feature, posb, pos_t, wr, wef, wep)


# speedup vs baseline: 1.3357x; 1.0003x over previous
import jax
import jax.numpy as jnp
from jax.experimental import pallas as pl

_BS = 16
_NPG = 4096
_C = 256
_NKP = 64
_KNN = 10


def _lane_min(x):
    w = x.shape[1]
    while w > 128:
        h = w // 2
        x = jnp.minimum(x[:, :h], x[:, h:])
        w = h
    return jnp.min(x, axis=1, keepdims=True)


def _row_max(x):
    r = x.shape[0]
    while r > 8:
        h = r // 2
        x = jnp.maximum(x[:h], x[h:])
        r = h
    return jnp.max(x, axis=0, keepdims=True)


def _row_sum(x):
    r = x.shape[0]
    while r > 8:
        h = r // 2
        x = x[:h] + x[h:]
        r = h
    return jnp.sum(x, axis=0, keepdims=True)


def _body(feat_ref, posb_ref, pos_t_ref, wr_ref, wef_ref, wep_ref, out_ref):
    feat = feat_ref[...].astype(jnp.bfloat16)
    posb = posb_ref[...]
    s = jnp.dot(feat, wr_ref[...], preferred_element_type=jnp.float32)
    m = _row_max(s)
    p = jnp.exp(s - m)
    pw = p / _row_sum(p)
    kp = jax.lax.dot_general(pw.astype(jnp.bfloat16), posb,
                             (((0,), (0,)), ((), ())),
                             preferred_element_type=jnp.float32)

    d2 = jnp.zeros((_NKP, _NPG), jnp.float32)
    for d in range(3):
        diff = kp[:, d:d + 1] - pos_t_ref[0, d:d + 1, :]
        d2 = d2 + diff * diff

    _CH = 8
    iot = jax.lax.broadcasted_iota(jnp.int32, (_CH, _NPG), 1).astype(
        jnp.float32)
    big = jnp.float32(jnp.inf)
    npgf = jnp.float32(_NPG)
    nch = _NKP // _CH
    d2s = [d2[c * _CH:(c + 1) * _CH] for c in range(nch)]
    dmins = [_lane_min(x) for x in d2s]
    hits = []
    for r in range(_KNN):
        for c in range(nch):
            idx = _lane_min(jnp.where(d2s[c] <= dmins[c], iot, npgf))
            hit = iot == idx
            hits.append(hit.astype(jnp.bfloat16))
            d2s[c] = jnp.where(hit, big, d2s[c])
            if r + 1 < _KNN:
                dmins[c] = _lane_min(d2s[c])

    onehot = jnp.concatenate(hits, axis=0)
    fpcat = jnp.concatenate([feat, posb], axis=1)
    g = jnp.dot(onehot, fpcat, preferred_element_type=jnp.float32)
    fj = g[:, :_C].astype(jnp.bfloat16)
    pj = g[:, _C:]
    kp_rep = jnp.concatenate([kp] * _KNN, axis=0)
    rel = (pj - kp_rep).astype(jnp.bfloat16)
    ext = (jnp.dot(fj, wef_ref[...], preferred_element_type=jnp.float32)
           + jnp.dot(rel, wep_ref[...], preferred_element_type=jnp.float32))
    ext = jnp.maximum(ext, 0.0).reshape(_KNN, _NKP, _C)
    out_ref[0] = jnp.sum(ext, axis=0) * (1.0 / _KNN)


def kernel(feature, pos, W_pool, W_regress, W_extract):
    del W_pool
    bf = jnp.bfloat16
    posb = pos.astype(bf)
    wr = W_regress[:_C].astype(bf)
    wef = W_extract[:_C].astype(bf)
    wep = W_extract[_C:].astype(bf)
    pos_t = pos.reshape(_BS, _NPG, 3).transpose(0, 2, 1)
    return pl.pallas_call(
        _body,
        grid=(_BS,),
        in_specs=[
            pl.BlockSpec((_NPG, _C), lambda b: (b, 0)),
            pl.BlockSpec((_NPG, 3), lambda b: (b, 0)),
            pl.BlockSpec((1, 3, _NPG), lambda b: (b, 0, 0)),
            pl.BlockSpec((_C, _NKP), lambda b: (0, 0)),
            pl.BlockSpec((_C, _C), lambda b: (0, 0)),
            pl.BlockSpec((3, _C), lambda b: (0, 0)),
        ],
        out_specs=pl.BlockSpec((1, _NKP, _C), lambda b: (b, 0, 0)),
        out_shape=jax.ShapeDtypeStruct((_BS, _NKP, _C), jnp.float32),
    )(feature, posb, pos_t, wr, wef, wep)
